# baseline (speedup 1.0000x reference)
import functools

import jax
import jax.numpy as jnp
import numpy as np
from jax import lax
from jax.experimental import pallas as pl
from jax.experimental.pallas import tpu as pltpu
from jax.experimental.pallas import tpu_sc as plsc

D_MODEL, D_K, D_V, N_Q = 1024, 128, 128, 64
B, T = 4, 2048
TH = T // 2
BT = B * T

_SC_INFO = plsc.get_sparse_core_info()
_NC, _NS, _L = _SC_INFO.num_cores, _SC_INFO.num_subcores, _SC_INFO.num_lanes
_NW = _NC * _NS
_TPW = BT // _NW


def _route_half(x, q_ref, wk_ref):
    k = jax.lax.dot_general(
        x, wk_ref[...], (((1,), (1,)), ((), ())),
        preferred_element_type=jnp.float32)
    attn = jax.lax.dot_general(
        k, q_ref[...], (((1,), (1,)), ((), ())),
        preferred_element_type=jnp.float32) * (1.0 / np.sqrt(N_Q))
    m = jnp.max(attn, axis=-1, keepdims=True)
    iota = jax.lax.broadcasted_iota(jnp.int32, attn.shape, 1)
    idx = jnp.min(jnp.where(attn == m, iota, N_Q), axis=-1, keepdims=True)
    onehot = (iota == idx).astype(jnp.float32)
    xsum = jax.lax.dot_general(
        onehot, x, (((0,), (0,)), ((), ())),
        preferred_element_type=jnp.float32)
    return idx, xsum


def _tc_body(xlo_ref, xhi_ref, q_ref, wk_ref, wv_ref, wfc_ref,
             out_ref, idx_ref):
    idx_lo, xsum_lo = _route_half(xlo_ref[0], q_ref, wk_ref)
    idx_hi, xsum_hi = _route_half(xhi_ref[0], q_ref, wk_ref)
    idx_ref[0, 0, :TH] = idx_lo.reshape(1, TH)[0]
    idx_ref[0, 0, TH:] = idx_hi.reshape(1, TH)[0]
    xsum = xsum_lo + xsum_hi
    vslot = jax.lax.dot_general(
        xsum, wv_ref[...], (((1,), (1,)), ((), ())),
        preferred_element_type=jnp.float32)
    out_ref[0] = jax.lax.dot_general(
        vslot, wfc_ref[...], (((1,), (1,)), ((), ())),
        preferred_element_type=jnp.float32)


def _tc_call(x, q, Wk, Wv, Wfc):
    return pl.pallas_call(
        _tc_body,
        grid=(B,),
        in_specs=[
            pl.BlockSpec((1, TH, D_MODEL), lambda b: (b, 0, 0)),
            pl.BlockSpec((1, TH, D_MODEL), lambda b: (b, 1, 0)),
            pl.BlockSpec((N_Q, D_K), lambda b: (0, 0)),
            pl.BlockSpec((D_K, D_MODEL), lambda b: (0, 0)),
            pl.BlockSpec((D_V, D_MODEL), lambda b: (0, 0)),
            pl.BlockSpec((D_MODEL, D_V), lambda b: (0, 0)),
        ],
        out_specs=[
            pl.BlockSpec((1, N_Q, D_MODEL), lambda b: (b, 0, 0)),
            pl.BlockSpec((1, 1, T), lambda b: (b, 0, 0)),
        ],
        out_shape=[
            jax.ShapeDtypeStruct((B, N_Q, D_MODEL), jnp.float32),
            jax.ShapeDtypeStruct((B, 1, T), jnp.int32),
        ],
        compiler_params=pltpu.CompilerParams(
            dimension_semantics=("arbitrary",),
        ),
    )(x, x, q, Wk, Wv, Wfc)


@functools.partial(
    pl.kernel,
    mesh=plsc.VectorSubcoreMesh(core_axis_name="c", subcore_axis_name="s"),
    out_type=jax.ShapeDtypeStruct((BT * N_Q,), jnp.float32),
    scratch_types=[
        pltpu.VMEM((_TPW,), jnp.int32),
        pltpu.VMEM((_TPW * N_Q,), jnp.float32),
    ],
    compiler_params=pltpu.CompilerParams(needs_layout_passes=False),
)
def _sc_onehot(idx_hbm, out_hbm, idx_v, rows_v):
    wid = lax.axis_index("s") * _NC + lax.axis_index("c")
    base = wid * _TPW
    pltpu.sync_copy(idx_hbm.at[pl.ds(base, _TPW)], idx_v)
    zeros = jnp.zeros((_L,), jnp.float32)

    def _zero_body(j, carry):
        rows_v[pl.ds(j * _L, _L)] = zeros
        return carry

    lax.fori_loop(0, (_TPW * N_Q) // _L, _zero_body, 0, unroll=8)

    ones = jnp.ones((_L,), jnp.float32)
    lane = lax.iota(jnp.int32, _L)

    def _scatter_body(g, carry):
        slot = idx_v[pl.ds(g * _L, _L)]
        flat = (g * _L + lane) * N_Q + slot
        plsc.store_scatter(rows_v, [flat], ones)
        return carry

    lax.fori_loop(0, _TPW // _L, _scatter_body, 0, unroll=4)

    pltpu.sync_copy(rows_v, out_hbm.at[pl.ds(base * N_Q, _TPW * N_Q)])


@jax.jit
def kernel(x, q, Wk, Wv, Wfc):
    out, idx = _tc_call(x, q, Wk, Wv, Wfc)
    hard_flat = _sc_onehot(idx.reshape(BT))
    return out, hard_flat.reshape(B, T, N_Q)


# speedup vs baseline: 1.5596x; 1.5596x over previous
import functools

import jax
import jax.numpy as jnp
import numpy as np
from jax.experimental import pallas as pl
from jax.experimental.pallas import tpu as pltpu

D_MODEL, D_K, D_V, N_Q = 1024, 128, 128, 64
B, T = 4, 2048
NS = 4
TQ = T // NS


def _route_part(x, q_ref, wk_ref):
    k = jax.lax.dot_general(
        x, wk_ref[...], (((1,), (1,)), ((), ())),
        preferred_element_type=jnp.float32)
    attn = jax.lax.dot_general(
        k, q_ref[...], (((1,), (1,)), ((), ())),
        preferred_element_type=jnp.float32) * (1.0 / np.sqrt(N_Q))
    m = jnp.max(attn, axis=-1, keepdims=True)
    iota = jax.lax.broadcasted_iota(jnp.int32, attn.shape, 1)
    idx = jnp.min(jnp.where(attn == m, iota, N_Q), axis=-1, keepdims=True)
    onehot = (iota == idx).astype(jnp.float32)
    xsum = jax.lax.dot_general(
        onehot, x, (((0,), (0,)), ((), ())),
        preferred_element_type=jnp.float32)
    return onehot, xsum


def _fused_body(*refs):
    x_refs = refs[:NS]
    q_ref, wk_ref, wv_ref, wfc_ref, out_ref, hard_ref = refs[NS:]
    xsum = None
    for s in range(NS):
        onehot, xs = _route_part(x_refs[s][0], q_ref, wk_ref)
        hard_ref[0, s * TQ:(s + 1) * TQ] = onehot
        xsum = xs if xsum is None else xsum + xs
    vslot = jax.lax.dot_general(
        xsum, wv_ref[...], (((1,), (1,)), ((), ())),
        preferred_element_type=jnp.float32)
    out_ref[0] = jax.lax.dot_general(
        vslot, wfc_ref[...], (((1,), (1,)), ((), ())),
        preferred_element_type=jnp.float32)


@functools.partial(jax.jit, static_argnames=("interpret",))
def kernel(x, q, Wk, Wv, Wfc, interpret=False):
    def _xspec(s):
        return pl.BlockSpec((1, TQ, D_MODEL), lambda b, s=s: (b, s, 0))

    out, hard = pl.pallas_call(
        _fused_body,
        grid=(B,),
        in_specs=[_xspec(s) for s in range(NS)] + [
            pl.BlockSpec((N_Q, D_K), lambda b: (0, 0)),
            pl.BlockSpec((D_K, D_MODEL), lambda b: (0, 0)),
            pl.BlockSpec((D_V, D_MODEL), lambda b: (0, 0)),
            pl.BlockSpec((D_MODEL, D_V), lambda b: (0, 0)),
        ],
        out_specs=[
            pl.BlockSpec((1, N_Q, D_MODEL), lambda b: (b, 0, 0)),
            pl.BlockSpec((1, T, N_Q), lambda b: (b, 0, 0)),
        ],
        out_shape=[
            jax.ShapeDtypeStruct((B, N_Q, D_MODEL), jnp.float32),
            jax.ShapeDtypeStruct((B, T, N_Q), jnp.float32),
        ],
        compiler_params=pltpu.CompilerParams(
            dimension_semantics=("arbitrary",),
        ),
        interpret=interpret,
    )(*([x] * NS), q, Wk, Wv, Wfc)
    return out, hard


# speedup vs baseline: 1.8089x; 1.1598x over previous
import jax
import jax.numpy as jnp
import numpy as np
from jax.experimental import pallas as pl
from jax.experimental.pallas import tpu as pltpu

D_MODEL, D_K, D_V, N_Q = 1024, 128, 128, 64
B, T = 4, 2048
TH = T // 2


def _route_half(x, q_ref, wk_ref):
    k = jax.lax.dot_general(
        x, wk_ref[...], (((1,), (1,)), ((), ())),
        preferred_element_type=jnp.float32)
    attn = jax.lax.dot_general(
        k, q_ref[...], (((1,), (1,)), ((), ())),
        preferred_element_type=jnp.float32) * (1.0 / np.sqrt(N_Q))
    m = jnp.max(attn, axis=-1, keepdims=True)
    iota = jax.lax.broadcasted_iota(jnp.int32, attn.shape, 1)
    idx = jnp.min(jnp.where(attn == m, iota, N_Q), axis=-1, keepdims=True)
    onehot = (iota == idx).astype(jnp.float32)
    xsum = jax.lax.dot_general(
        onehot, x, (((0,), (0,)), ((), ())),
        preferred_element_type=jnp.float32)
    return onehot, xsum


def _fused_body(xlo_ref, xhi_ref, q_ref, wk_ref, wv_ref, wfc_ref,
                out_ref, hard_ref):
    onehot_lo, xsum_lo = _route_half(xlo_ref[0], q_ref, wk_ref)
    hard_ref[0, :TH] = onehot_lo
    onehot_hi, xsum_hi = _route_half(xhi_ref[0], q_ref, wk_ref)
    hard_ref[0, TH:] = onehot_hi
    xsum = xsum_lo + xsum_hi
    vslot = jax.lax.dot_general(
        xsum, wv_ref[...], (((1,), (1,)), ((), ())),
        preferred_element_type=jnp.float32)
    out_ref[0] = jax.lax.dot_general(
        vslot, wfc_ref[...], (((1,), (1,)), ((), ())),
        preferred_element_type=jnp.float32)


@jax.jit
def kernel(x, q, Wk, Wv, Wfc):
    out, hard = pl.pallas_call(
        _fused_body,
        grid=(B,),
        in_specs=[
            pl.BlockSpec((1, TH, D_MODEL), lambda b: (b, 0, 0)),
            pl.BlockSpec((1, TH, D_MODEL), lambda b: (b, 1, 0)),
            pl.BlockSpec((N_Q, D_K), lambda b: (0, 0)),
            pl.BlockSpec((D_K, D_MODEL), lambda b: (0, 0)),
            pl.BlockSpec((D_V, D_MODEL), lambda b: (0, 0)),
            pl.BlockSpec((D_MODEL, D_V), lambda b: (0, 0)),
        ],
        out_specs=[
            pl.BlockSpec((1, N_Q, D_MODEL), lambda b: (b, 0, 0)),
            pl.BlockSpec((1, T, N_Q), lambda b: (b, 0, 0)),
        ],
        out_shape=[
            jax.ShapeDtypeStruct((B, N_Q, D_MODEL), jnp.float32),
            jax.ShapeDtypeStruct((B, T, N_Q), jnp.float32),
        ],
        compiler_params=pltpu.CompilerParams(
            dimension_semantics=("arbitrary",),
        ),
    )(x, x, q, Wk, Wv, Wfc)
    return out, hard
